# Initial kernel scaffold; baseline (speedup 1.0000x reference)
import jax
import jax.numpy as jnp
from jax.experimental import pallas as pl


def kernel(queries, keys, values):
    raise NotImplementedError("write your pallas kernel here")


# speedup vs baseline: 2.0408x; 2.0408x over previous
import jax
import jax.numpy as jnp
from jax.experimental import pallas as pl
from jax.experimental.pallas import tpu as pltpu

_L = 128
_H = 8
_U = 25

_NEG = -3.0e38
_POS = 3.0e38


def _body(q_ref, k_ref, v_ref, cnt_ref, out_ref):
    q = q_ref[...]
    k = k_ref[...]
    v = v_ref[...]
    cnt = cnt_ref[...]
    member = cnt > 0.0

    sumsel = jnp.sum(k * cnt, axis=1, keepdims=True)
    maxsel = jnp.max(jnp.where(member, k, _NEG), axis=1, keepdims=True)
    minsel = jnp.min(jnp.where(member, k, _POS), axis=1, keepdims=True)
    a = maxsel - sumsel * (1.0 / _L)
    b = minsel - sumsel * (1.0 / _L)
    m = jnp.where(q >= 0.0, q * a, q * b)

    lane = jax.lax.broadcasted_iota(jnp.int32, (_H, _L), 1).astype(jnp.float32)

    def step(_, carry):
        active, selected = carry
        act = active > 0.0
        cur = jnp.max(jnp.where(act, m, _NEG), axis=1, keepdims=True)
        is_max = act & (m == cur)
        first = jnp.min(jnp.where(is_max, lane, _POS), axis=1, keepdims=True)
        pick = (lane == first).astype(jnp.float32)
        return active * (1.0 - pick), jnp.maximum(selected, pick)

    active0 = jnp.ones((_H, _L), jnp.float32)
    sel0 = jnp.zeros((_H, _L), jnp.float32)
    _, selected_f = jax.lax.fori_loop(0, _U, step, (active0, sel0))
    selected = selected_f > 0.0

    row = jax.lax.broadcasted_iota(jnp.int32, (_L, _L), 0)
    col = jax.lax.broadcasted_iota(jnp.int32, (_L, _L), 1)
    tri = (row <= col).astype(jnp.float32)
    pref = jnp.dot(v, tri, preferred_element_type=jnp.float32)

    out_ref[...] = jnp.where(selected, pref / (lane + 1.0), v)


def kernel(queries, keys, values):
    q = queries.reshape(_H, _L)
    k = keys.reshape(_H, _L)
    v = values.reshape(_H, _L)

    _, rk2 = jax.random.split(jax.random.key(1))
    idx_k = jax.random.randint(rk2, (_U,), 0, _L)
    cnt = jnp.zeros((_L,), jnp.float32).at[idx_k].add(1.0)
    cnt = jnp.broadcast_to(cnt[None, :], (_H, _L))

    out = pl.pallas_call(
        _body,
        out_shape=jax.ShapeDtypeStruct((_H, _L), jnp.float32),
        in_specs=[
            pl.BlockSpec((_H, _L), lambda: (0, 0)),
            pl.BlockSpec((_H, _L), lambda: (0, 0)),
            pl.BlockSpec((_H, _L), lambda: (0, 0)),
            pl.BlockSpec((_H, _L), lambda: (0, 0)),
        ],
        out_specs=pl.BlockSpec((_H, _L), lambda: (0, 0)),
    )(q, k, v, cnt)

    return out.reshape(1, _H, _L, 1)


# speedup vs baseline: 5.0940x; 2.4961x over previous
import jax
import jax.numpy as jnp
from jax.experimental import pallas as pl
from jax.experimental.pallas import tpu as pltpu

_L = 128
_H = 8
_U = 25

_NEG = -3.0e38
_POS = 3.0e38

_IDX_K = (11, 16, 17, 21, 23, 26, 28, 30, 53, 55, 60, 69, 70, 77,
          85, 91, 96, 100, 103, 104, 109, 110, 114, 116)
_DUP_IDX = 60


def _body(q_ref, k_ref, v_ref, out_ref):
    q = q_ref[...]
    k = k_ref[...]
    v = v_ref[...]

    lane_i = jax.lax.broadcasted_iota(jnp.int32, (_H, _L), 1)
    member = lane_i == _IDX_K[0]
    for ix in _IDX_K[1:]:
        member = member | (lane_i == ix)
    cnt = member.astype(jnp.float32) + (lane_i == _DUP_IDX).astype(jnp.float32)

    sumsel = jnp.sum(k * cnt, axis=1, keepdims=True)
    maxsel = jnp.max(jnp.where(member, k, _NEG), axis=1, keepdims=True)
    minsel = jnp.min(jnp.where(member, k, _POS), axis=1, keepdims=True)
    a = maxsel - sumsel * (1.0 / _L)
    b = minsel - sumsel * (1.0 / _L)
    m = jnp.where(q >= 0.0, q * a, q * b)

    lane = jax.lax.broadcasted_iota(jnp.int32, (_H, _L), 1).astype(jnp.float32)

    def step(_, carry):
        active, selected = carry
        act = active > 0.0
        cur = jnp.max(jnp.where(act, m, _NEG), axis=1, keepdims=True)
        is_max = act & (m == cur)
        first = jnp.min(jnp.where(is_max, lane, _POS), axis=1, keepdims=True)
        pick = (lane == first).astype(jnp.float32)
        return active * (1.0 - pick), jnp.maximum(selected, pick)

    active0 = jnp.ones((_H, _L), jnp.float32)
    sel0 = jnp.zeros((_H, _L), jnp.float32)
    _, selected_f = jax.lax.fori_loop(0, _U, step, (active0, sel0))
    selected = selected_f > 0.0

    row = jax.lax.broadcasted_iota(jnp.int32, (_L, _L), 0)
    col = jax.lax.broadcasted_iota(jnp.int32, (_L, _L), 1)
    tri = (row <= col).astype(jnp.float32)
    pref = jnp.dot(v, tri, preferred_element_type=jnp.float32)

    out_ref[...] = jnp.where(selected, pref / (lane + 1.0), v)


def kernel(queries, keys, values):
    q = queries.reshape(_H, _L)
    k = keys.reshape(_H, _L)
    v = values.reshape(_H, _L)

    out = pl.pallas_call(
        _body,
        out_shape=jax.ShapeDtypeStruct((_H, _L), jnp.float32),
        in_specs=[
            pl.BlockSpec((_H, _L), lambda: (0, 0)),
            pl.BlockSpec((_H, _L), lambda: (0, 0)),
            pl.BlockSpec((_H, _L), lambda: (0, 0)),
        ],
        out_specs=pl.BlockSpec((_H, _L), lambda: (0, 0)),
    )(q, k, v)

    return out.reshape(1, _H, _L, 1)


# speedup vs baseline: 8.8575x; 1.7388x over previous
import jax
import jax.numpy as jnp
from jax.experimental import pallas as pl
from jax.experimental.pallas import tpu as pltpu

_L = 128
_H = 8
_U = 25

_NEG = -3.0e38
_POS = 3.0e38

_IDX_K = (11, 16, 17, 21, 23, 26, 28, 30, 53, 55, 60, 69, 70, 77,
          85, 91, 96, 100, 103, 104, 109, 110, 114, 116)
_DUP_IDX = 60


def _body(q_ref, k_ref, v_ref, out_ref):
    q = q_ref[...]
    k = k_ref[...]
    v = v_ref[...]

    lane_i = jax.lax.broadcasted_iota(jnp.int32, (_H, _L), 1)
    member = lane_i == _IDX_K[0]
    for ix in _IDX_K[1:]:
        member = member | (lane_i == ix)
    cnt = member.astype(jnp.float32) + (lane_i == _DUP_IDX).astype(jnp.float32)

    sumsel = jnp.sum(k * cnt, axis=1, keepdims=True)
    maxsel = jnp.max(jnp.where(member, k, _NEG), axis=1, keepdims=True)
    minsel = jnp.min(jnp.where(member, k, _POS), axis=1, keepdims=True)
    a = maxsel - sumsel * (1.0 / _L)
    b = minsel - sumsel * (1.0 / _L)
    m = jnp.where(q >= 0.0, q * a, q * b)

    lane = jax.lax.broadcasted_iota(jnp.int32, (_H, _L), 1).astype(jnp.float32)

    mpad = jnp.concatenate([m, jnp.zeros((_L - _H, _L), jnp.float32)], axis=0)
    mt = mpad.T

    row = jax.lax.broadcasted_iota(jnp.int32, (_L, _L), 0)
    col = jax.lax.broadcasted_iota(jnp.int32, (_L, _L), 1)
    idx_lt = row < col

    sel_rows = []
    for h in range(_H):
        col_h = jnp.broadcast_to(mt[:, h:h + 1], (_L, _L))
        row_h = jnp.broadcast_to(m[h:h + 1, :], (_L, _L))
        beats = (col_h > row_h) | ((col_h == row_h) & idx_lt)
        rank = jnp.sum(beats.astype(jnp.float32), axis=0, keepdims=True)
        sel_rows.append(rank)
    selected = jnp.concatenate(sel_rows, axis=0) < float(_U)

    tri = (row <= col).astype(jnp.float32)
    pref = jnp.dot(v, tri, preferred_element_type=jnp.float32)

    out_ref[...] = jnp.where(selected, pref / (lane + 1.0), v)


def kernel(queries, keys, values):
    q = queries.reshape(_H, _L)
    k = keys.reshape(_H, _L)
    v = values.reshape(_H, _L)

    out = pl.pallas_call(
        _body,
        out_shape=jax.ShapeDtypeStruct((_H, _L), jnp.float32),
        in_specs=[
            pl.BlockSpec((_H, _L), lambda: (0, 0)),
            pl.BlockSpec((_H, _L), lambda: (0, 0)),
            pl.BlockSpec((_H, _L), lambda: (0, 0)),
        ],
        out_specs=pl.BlockSpec((_H, _L), lambda: (0, 0)),
    )(q, k, v)

    return out.reshape(1, _H, _L, 1)
